# baseline (speedup 1.0000x reference)
import jax
import jax.numpy as jnp
from jax import lax
from jax.experimental import pallas as pl
from jax.experimental.pallas import tpu as pltpu
from jax.experimental.pallas import tpu_sc as plsc


_ROWS = 8192
_COLS = 1024
_NC = 2
_NS = 16
_NW = _NC * _NS
_ROWS_PER_W = _ROWS // _NW
_CHUNK = 32
_N_CHUNKS = _ROWS_PER_W // _CHUNK


def _sc_copy(x_hbm, out_hbm, spmem, lsem, ssem):
    wid = lax.axis_index("s") * _NC + lax.axis_index("c")
    sid = lax.axis_index("s")
    base = wid * _ROWS_PER_W

    def load(i):
        return pltpu.make_async_copy(
            x_hbm.at[pl.ds(base + i * _CHUNK, _CHUNK), :],
            spmem.at[sid, i % 2],
            lsem.at[i % 2],
        )

    def store(i):
        return pltpu.make_async_copy(
            spmem.at[sid, i % 2],
            out_hbm.at[pl.ds(base + i * _CHUNK, _CHUNK), :],
            ssem.at[i % 2],
        )

    load(0).start()
    for i in range(_N_CHUNKS):
        if i + 1 < _N_CHUNKS:
            if i - 1 >= 0:
                store(i - 1).wait()
            load(i + 1).start()
        load(i).wait()
        store(i).start()
    store(_N_CHUNKS - 2).wait()
    store(_N_CHUNKS - 1).wait()


def kernel(x):
    mesh = plsc.VectorSubcoreMesh(core_axis_name="c", subcore_axis_name="s")
    gathered = pl.kernel(
        _sc_copy,
        out_type=jax.ShapeDtypeStruct((_ROWS, _COLS), jnp.float32),
        mesh=mesh,
        scratch_types=[
            pltpu.VMEM_SHARED((_NS, 2, _CHUNK, _COLS), jnp.float32),
            pltpu.SemaphoreType.DMA((2,)),
            pltpu.SemaphoreType.DMA((2,)),
        ],
    )(x)
    sizes = jnp.array([_ROWS], dtype=jnp.int32)
    return (gathered, sizes)
